# baseline (speedup 1.0000x reference)
import functools

import jax
import jax.numpy as jnp
from jax import lax
from jax.experimental import pallas as pl
from jax.experimental.pallas import tpu as pltpu
from jax.experimental.pallas import tpu_sc as plsc

BEV_H = 200
BEV_W = 176
HW = BEV_H * BEV_W
IN_CH = 128
OUT_CH = 256
B = 4
NSEG = B * HW
NPTS = 120000
NW = 32
CPW = 4480
NSEG_PAD = NW * CPW
NSUB = 8
SCELLS = CPW // NSUB
CHUNK = 4000
NCHUNKS = NPTS // CHUNK
LIST_CAP = 16384
SUB_CAP = 2048
GROUP = 128
SENT = -3.0e38


def _sc_scatter(flat, features):
    mesh = plsc.VectorSubcoreMesh(core_axis_name="c", subcore_axis_name="s")

    @functools.partial(
        pl.kernel,
        out_type=jax.ShapeDtypeStruct((NSEG_PAD * IN_CH,), jnp.float32),
        mesh=mesh,
        compiler_params=pltpu.CompilerParams(needs_layout_passes=False),
        scratch_types=[
            pltpu.VMEM((CHUNK,), jnp.int32),
            pltpu.VMEM((LIST_CAP + 16,), jnp.int32),
            pltpu.VMEM((SUB_CAP + 16,), jnp.int32),
            pltpu.VMEM((16, IN_CH), jnp.float32),
            pltpu.VMEM((SCELLS * IN_CH,), jnp.float32),
            pltpu.SemaphoreType.DMA,
        ],
    )
    def k(flat_hbm, feat_hbm, bev_hbm, chunkbuf, listbuf, sublist,
          stage, slab, sem0):
        wid = lax.axis_index("s") * 2 + lax.axis_index("c")
        lo = wid * CPW
        iota = lax.iota(jnp.int32, 16)
        zeros16 = jnp.zeros((16,), jnp.int32)
        sent16 = jnp.full((16,), SENT, jnp.float32)

        def zl(i, _):
            listbuf[pl.ds(i * 16, 16)] = zeros16
            return 0
        lax.fori_loop(0, (LIST_CAP + 16) // 16, zl, 0)

        n = jnp.int32(0)
        for c in range(NCHUNKS):
            pltpu.sync_copy(flat_hbm.at[pl.ds(c * CHUNK, CHUNK)], chunkbuf)

            def scan(i, n, c=c):
                v = chunkbuf[pl.ds(i * 16, 16)]
                rel = v - lo
                m = (rel >= 0) & (rel < CPW)
                pid = (c * CHUNK) + i * 16 + iota
                rec = (pid << 13) | (rel & 8191)
                cs = plsc.cumsum(m.astype(jnp.int32))
                pos = jnp.where(m, n + cs - 1, LIST_CAP)
                plsc.store_scatter(listbuf, [pos], rec, mask=m)
                return jnp.minimum(n + cs[15], jnp.int32(LIST_CAP))
            n = lax.fori_loop(0, CHUNK // 16, scan, n)

        nvec = (n + 15) // 16
        for s in range(NSUB):
            def init(i, _):
                slab[pl.ds(i * 16, 16)] = sent16
                return 0
            lax.fori_loop(0, SCELLS * IN_CH // 16, init, 0)

            def filt(i, m, s=s):
                rec = listbuf[pl.ds(i * 16, 16)]
                rel = rec & 8191
                inm = ((rel >= s * SCELLS) & (rel < (s + 1) * SCELLS)
                       & (i * 16 + iota < n))
                cs = plsc.cumsum(inm.astype(jnp.int32))
                pos = jnp.where(inm, m + cs - 1, SUB_CAP)
                plsc.store_scatter(sublist, [pos], rec, mask=inm)
                return jnp.minimum(m + cs[15], jnp.int32(SUB_CAP))
            m = lax.fori_loop(0, nvec, filt, jnp.int32(0))

            ngroups = (m + 15) // 16

            def group_body(g, _, s=s):
                gbase = g * 16
                rec16 = sublist[pl.ds(gbase, 16)]
                idx = jnp.where(gbase + iota < m, rec16 >> 13, 0)
                pltpu.async_copy(feat_hbm.at[idx], stage, sem0).wait()

                def rmw(r, _):
                    rec = sublist[pl.ds(gbase + r, 16)][0]
                    sbase = ((rec & 8191) - s * SCELLS) * IN_CH
                    for v in range(IN_CH // 16):
                        cur = slab[pl.ds(sbase + v * 16, 16)]
                        fv = stage[r, pl.ds(v * 16, 16)]
                        slab[pl.ds(sbase + v * 16, 16)] = jnp.maximum(cur, fv)
                    return 0
                lax.fori_loop(0, jnp.minimum(m - gbase, 16), rmw, 0)
                return 0
            lax.fori_loop(0, ngroups, group_body, 0)

            pltpu.sync_copy(
                slab,
                bev_hbm.at[pl.ds((lo + s * SCELLS) * IN_CH, SCELLS * IN_CH)])

    return k(flat, features)


def _conv_bn_relu_body(x_ref, w_ref, s_ref, b_ref, o_ref):
    x = x_ref[...]
    x = jnp.where(x > -1.0e38, x, 0.0)
    w = w_ref[...]
    acc = jax.lax.dot_general(w, x, (((1,), (1,)), ((), ())),
                              preferred_element_type=jnp.float32)
    o_ref[0] = jnp.maximum(acc * s_ref[...] + b_ref[...], 0.0)


def _conv_bn_relu(bev_pad, W, scale2, beta2):
    TS = 3200
    nt = HW // TS
    return pl.pallas_call(
        _conv_bn_relu_body,
        grid=(B, nt),
        in_specs=[
            pl.BlockSpec((TS, IN_CH), lambda b, t: (b * nt + t, 0)),
            pl.BlockSpec((OUT_CH, IN_CH), lambda b, t: (0, 0)),
            pl.BlockSpec((OUT_CH, 1), lambda b, t: (0, 0)),
            pl.BlockSpec((OUT_CH, 1), lambda b, t: (0, 0)),
        ],
        out_specs=pl.BlockSpec((1, OUT_CH, TS), lambda b, t: (b, 0, t)),
        out_shape=jax.ShapeDtypeStruct((B, OUT_CH, HW), jnp.float32),
    )(bev_pad, W, scale2, beta2)


def kernel(features, coordinates, batch_size, W, gamma, beta):
    flat = (coordinates[:, 0] * HW + coordinates[:, 2] * BEV_W
            + coordinates[:, 3])
    bev_pad = _sc_scatter(flat, features).reshape(NSEG_PAD, IN_CH)
    scale2 = (gamma / jnp.sqrt(1.0 + 1e-5)).reshape(OUT_CH, 1)
    beta2 = beta.reshape(OUT_CH, 1)
    out = _conv_bn_relu(bev_pad, W, scale2, beta2)
    return out.reshape(B, OUT_CH, BEV_H, BEV_W)


# speedup vs baseline: 1.1785x; 1.1785x over previous
import functools

import jax
import jax.numpy as jnp
from jax import lax
from jax.experimental import pallas as pl
from jax.experimental.pallas import tpu as pltpu
from jax.experimental.pallas import tpu_sc as plsc

BEV_H = 200
BEV_W = 176
HW = BEV_H * BEV_W
IN_CH = 128
OUT_CH = 256
B = 4
NSEG = B * HW
NPTS = 120000
NW = 32
CPW = 4480
NSEG_PAD = NW * CPW
NSUB = 8
SCELLS = CPW // NSUB
CHUNK = 4000
NCHUNKS = NPTS // CHUNK
LIST_CAP = 16384
SUB_CAP = 2048
GROUP = 128
SENT = -3.0e38


def _sc_scatter(flat, features):
    mesh = plsc.VectorSubcoreMesh(core_axis_name="c", subcore_axis_name="s")

    @functools.partial(
        pl.kernel,
        out_type=jax.ShapeDtypeStruct((NSEG_PAD, IN_CH), jnp.float32),
        mesh=mesh,
        compiler_params=pltpu.CompilerParams(needs_layout_passes=False),
        scratch_types=[
            pltpu.VMEM((CHUNK,), jnp.int32),
            pltpu.VMEM((LIST_CAP + 16,), jnp.int32),
            pltpu.VMEM((SUB_CAP + 16,), jnp.int32),
            pltpu.VMEM((16, IN_CH), jnp.float32),
            pltpu.VMEM((SCELLS, IN_CH), jnp.float32),
            pltpu.SemaphoreType.DMA,
        ],
    )
    def k(flat_hbm, feat_hbm, bev_hbm, chunkbuf, listbuf, sublist,
          stage, slab, sem0):
        wid = lax.axis_index("s") * 2 + lax.axis_index("c")
        lo = wid * CPW
        iota = lax.iota(jnp.int32, 16)
        zeros16 = jnp.zeros((16,), jnp.int32)
        sent16 = jnp.full((16,), SENT, jnp.float32)

        def zl(i, _):
            listbuf[pl.ds(i * 16, 16)] = zeros16
            return 0
        lax.fori_loop(0, (LIST_CAP + 16) // 16, zl, 0)

        n = jnp.int32(0)
        for c in range(NCHUNKS):
            pltpu.sync_copy(flat_hbm.at[pl.ds(c * CHUNK, CHUNK)], chunkbuf)

            def scan(i, n, c=c):
                v = chunkbuf[pl.ds(i * 16, 16)]
                rel = v - lo
                m = (rel >= 0) & (rel < CPW)
                pid = (c * CHUNK) + i * 16 + iota
                rec = (pid << 13) | (rel & 8191)
                cs = plsc.cumsum(m.astype(jnp.int32))
                pos = jnp.where(m, n + cs - 1, LIST_CAP)
                plsc.store_scatter(listbuf, [pos], rec, mask=m)
                return jnp.minimum(n + cs[15], jnp.int32(LIST_CAP))
            n = lax.fori_loop(0, CHUNK // 16, scan, n)

        nvec = (n + 15) // 16
        for s in range(NSUB):
            def init(i, _):
                for v in range(IN_CH // 16):
                    slab[i, pl.ds(v * 16, 16)] = sent16
                return 0
            lax.fori_loop(0, SCELLS, init, 0)

            def filt(i, m, s=s):
                rec = listbuf[pl.ds(i * 16, 16)]
                rel = rec & 8191
                inm = ((rel >= s * SCELLS) & (rel < (s + 1) * SCELLS)
                       & (i * 16 + iota < n))
                cs = plsc.cumsum(inm.astype(jnp.int32))
                pos = jnp.where(inm, m + cs - 1, SUB_CAP)
                plsc.store_scatter(sublist, [pos], rec, mask=inm)
                return jnp.minimum(m + cs[15], jnp.int32(SUB_CAP))
            m = lax.fori_loop(0, nvec, filt, jnp.int32(0))

            ngroups = (m + 15) // 16

            def group_body(g, _, s=s):
                gbase = g * 16
                rec16 = sublist[pl.ds(gbase, 16)]
                idx = jnp.where(gbase + iota < m, rec16 >> 13, 0)
                pltpu.async_copy(feat_hbm.at[idx], stage, sem0).wait()

                def rmw(r, _):
                    rec = sublist[pl.ds(gbase + r, 16)][0]
                    srel = (rec & 8191) - s * SCELLS
                    for v in range(IN_CH // 16):
                        cur = slab[srel, pl.ds(v * 16, 16)]
                        fv = stage[r, pl.ds(v * 16, 16)]
                        slab[srel, pl.ds(v * 16, 16)] = jnp.maximum(cur, fv)
                    return 0
                lax.fori_loop(0, jnp.minimum(m - gbase, 16), rmw, 0)
                return 0
            lax.fori_loop(0, ngroups, group_body, 0)

            pltpu.sync_copy(
                slab, bev_hbm.at[pl.ds(lo + s * SCELLS, SCELLS), :])

    return k(flat, features)


def _conv_bn_relu_body(x_ref, w_ref, s_ref, b_ref, o_ref):
    x = x_ref[...]
    x = jnp.where(x > -1.0e38, x, 0.0)
    w = w_ref[...]
    acc = jax.lax.dot_general(w, x, (((1,), (1,)), ((), ())),
                              preferred_element_type=jnp.float32)
    o_ref[0] = jnp.maximum(acc * s_ref[...] + b_ref[...], 0.0)


def _conv_bn_relu(bev_pad, W, scale2, beta2):
    TS = 3200
    nt = HW // TS
    return pl.pallas_call(
        _conv_bn_relu_body,
        grid=(B, nt),
        in_specs=[
            pl.BlockSpec((TS, IN_CH), lambda b, t: (b * nt + t, 0)),
            pl.BlockSpec((OUT_CH, IN_CH), lambda b, t: (0, 0)),
            pl.BlockSpec((OUT_CH, 1), lambda b, t: (0, 0)),
            pl.BlockSpec((OUT_CH, 1), lambda b, t: (0, 0)),
        ],
        out_specs=pl.BlockSpec((1, OUT_CH, TS), lambda b, t: (b, 0, t)),
        out_shape=jax.ShapeDtypeStruct((B, OUT_CH, HW), jnp.float32),
    )(bev_pad, W, scale2, beta2)


def kernel(features, coordinates, batch_size, W, gamma, beta):
    flat = (coordinates[:, 0] * HW + coordinates[:, 2] * BEV_W
            + coordinates[:, 3])
    bev_pad = _sc_scatter(flat, features)
    scale2 = (gamma / jnp.sqrt(1.0 + 1e-5)).reshape(OUT_CH, 1)
    beta2 = beta.reshape(OUT_CH, 1)
    out = _conv_bn_relu(bev_pad, W, scale2, beta2)
    return out.reshape(B, OUT_CH, BEV_H, BEV_W)


# speedup vs baseline: 1.2567x; 1.0664x over previous
import functools

import jax
import jax.numpy as jnp
from jax import lax
from jax.experimental import pallas as pl
from jax.experimental.pallas import tpu as pltpu
from jax.experimental.pallas import tpu_sc as plsc

BEV_H = 200
BEV_W = 176
HW = BEV_H * BEV_W
IN_CH = 128
OUT_CH = 256
B = 4
NSEG = B * HW
NPTS = 120000
NW = 32
CPW = 4480
NSEG_PAD = NW * CPW
NSUB = 8
SCELLS = CPW // NSUB
CHUNK = 4000
NCHUNKS = NPTS // CHUNK
LIST_CAP = 16384
SUB_CAP = 2048
GROUP = 128
SENT = -3.0e38


def _sc_scatter(flat, features):
    mesh = plsc.VectorSubcoreMesh(core_axis_name="c", subcore_axis_name="s")

    @functools.partial(
        pl.kernel,
        out_type=jax.ShapeDtypeStruct((NSEG_PAD, IN_CH), jnp.float32),
        mesh=mesh,
        compiler_params=pltpu.CompilerParams(needs_layout_passes=False),
        scratch_types=[
            pltpu.VMEM((CHUNK,), jnp.int32),
            pltpu.VMEM((LIST_CAP + 16,), jnp.int32),
            pltpu.VMEM((SUB_CAP + 48,), jnp.int32),
            pltpu.VMEM((16, IN_CH), jnp.float32),
            pltpu.VMEM((16, IN_CH), jnp.float32),
            pltpu.VMEM((SCELLS, IN_CH), jnp.float32),
            pltpu.SemaphoreType.DMA,
            pltpu.SemaphoreType.DMA,
        ],
    )
    def k(flat_hbm, feat_hbm, bev_hbm, chunkbuf, listbuf, sublist,
          stage_a, stage_b, slab, sem0, sem1):
        wid = lax.axis_index("s") * 2 + lax.axis_index("c")
        lo = wid * CPW
        iota = lax.iota(jnp.int32, 16)
        zeros16 = jnp.zeros((16,), jnp.int32)
        sent16 = jnp.full((16,), SENT, jnp.float32)

        def zl(i, _):
            listbuf[pl.ds(i * 16, 16)] = zeros16
            return 0
        lax.fori_loop(0, (LIST_CAP + 16) // 16, zl, 0)

        n = jnp.int32(0)
        for c in range(NCHUNKS):
            pltpu.sync_copy(flat_hbm.at[pl.ds(c * CHUNK, CHUNK)], chunkbuf)

            def scan(i, n, c=c):
                v = chunkbuf[pl.ds(i * 16, 16)]
                rel = v - lo
                m = (rel >= 0) & (rel < CPW)
                pid = (c * CHUNK) + i * 16 + iota
                rec = (pid << 13) | (rel & 8191)
                cs = plsc.cumsum(m.astype(jnp.int32))
                pos = jnp.where(m, n + cs - 1, LIST_CAP)
                plsc.store_scatter(listbuf, [pos], rec, mask=m)
                return jnp.minimum(n + cs[15], jnp.int32(LIST_CAP))
            n = lax.fori_loop(0, CHUNK // 16, scan, n)

        nvec = (n + 15) // 16
        for s in range(NSUB):
            def init(i, _):
                for v in range(IN_CH // 16):
                    slab[i, pl.ds(v * 16, 16)] = sent16
                return 0
            lax.fori_loop(0, SCELLS, init, 0)

            def filt(i, m, s=s):
                rec = listbuf[pl.ds(i * 16, 16)]
                rel = rec & 8191
                inm = ((rel >= s * SCELLS) & (rel < (s + 1) * SCELLS)
                       & (i * 16 + iota < n))
                cs = plsc.cumsum(inm.astype(jnp.int32))
                pos = jnp.where(inm, m + cs - 1, SUB_CAP)
                plsc.store_scatter(sublist, [pos], rec, mask=inm)
                return jnp.minimum(m + cs[15], jnp.int32(SUB_CAP))
            m = lax.fori_loop(0, nvec, filt, jnp.int32(0))

            ngroups = (m + 31) // 32

            def group_body(g, _, s=s):
                gbase = g * 32
                rec_a = sublist[pl.ds(gbase, 16)]
                idx_a = jnp.where(gbase + iota < m, rec_a >> 13, 0)
                ha = pltpu.async_copy(feat_hbm.at[idx_a], stage_a, sem0)
                rec_b = sublist[pl.ds(gbase + 16, 16)]
                idx_b = jnp.where(gbase + 16 + iota < m, rec_b >> 13, 0)
                hb = pltpu.async_copy(feat_hbm.at[idx_b], stage_b, sem1)

                def mk_rmw(stage, base):
                    def rmw(r, _):
                        rec = sublist[pl.ds(base + r, 16)][0]
                        srel = (rec & 8191) - s * SCELLS
                        for v in range(IN_CH // 16):
                            cur = slab[srel, pl.ds(v * 16, 16)]
                            fv = stage[r, pl.ds(v * 16, 16)]
                            slab[srel, pl.ds(v * 16, 16)] = jnp.maximum(cur, fv)
                        return 0
                    return rmw
                ha.wait()
                lax.fori_loop(0, jnp.minimum(m - gbase, 16),
                              mk_rmw(stage_a, gbase), 0)
                hb.wait()
                lax.fori_loop(0, jnp.maximum(jnp.minimum(m - gbase - 16, 16), 0),
                              mk_rmw(stage_b, gbase + 16), 0)
                return 0
            lax.fori_loop(0, ngroups, group_body, 0)

            pltpu.sync_copy(
                slab, bev_hbm.at[pl.ds(lo + s * SCELLS, SCELLS), :])

    return k(flat, features)


def _conv_bn_relu_body(x_ref, w_ref, s_ref, b_ref, o_ref):
    x = x_ref[...]
    x = jnp.where(x > -1.0e38, x, 0.0)
    w = w_ref[...]
    acc = jax.lax.dot_general(w, x, (((1,), (1,)), ((), ())),
                              preferred_element_type=jnp.float32)
    o_ref[0] = jnp.maximum(acc * s_ref[...] + b_ref[...], 0.0)


def _conv_bn_relu(bev_pad, W, scale2, beta2):
    TS = 3200
    nt = HW // TS
    return pl.pallas_call(
        _conv_bn_relu_body,
        grid=(B, nt),
        in_specs=[
            pl.BlockSpec((TS, IN_CH), lambda b, t: (b * nt + t, 0)),
            pl.BlockSpec((OUT_CH, IN_CH), lambda b, t: (0, 0)),
            pl.BlockSpec((OUT_CH, 1), lambda b, t: (0, 0)),
            pl.BlockSpec((OUT_CH, 1), lambda b, t: (0, 0)),
        ],
        out_specs=pl.BlockSpec((1, OUT_CH, TS), lambda b, t: (b, 0, t)),
        out_shape=jax.ShapeDtypeStruct((B, OUT_CH, HW), jnp.float32),
    )(bev_pad, W, scale2, beta2)


def kernel(features, coordinates, batch_size, W, gamma, beta):
    flat = (coordinates[:, 0] * HW + coordinates[:, 2] * BEV_W
            + coordinates[:, 3])
    bev_pad = _sc_scatter(flat, features)
    scale2 = (gamma / jnp.sqrt(1.0 + 1e-5)).reshape(OUT_CH, 1)
    beta2 = beta.reshape(OUT_CH, 1)
    out = _conv_bn_relu(bev_pad, W, scale2, beta2)
    return out.reshape(B, OUT_CH, BEV_H, BEV_W)
